# baseline (speedup 1.0000x reference)
import functools
import jax
import jax.numpy as jnp
from jax import lax
from jax.experimental import pallas as pl
from jax.experimental.pallas import tpu as pltpu
from jax.experimental.pallas import tpu_sc as plsc

EMB = 64
SCALE = 8.0
NC = 2
NS = 16
NW = NC * NS
G = 128


@functools.partial(jax.jit, static_argnums=(2,))
def _run(tok, table, ng):
    mesh = plsc.VectorSubcoreMesh(core_axis_name="c", subcore_axis_name="s")

    @functools.partial(
        pl.kernel,
        mesh=mesh,
        out_type=jax.ShapeDtypeStruct((NW, ng, G, EMB), jnp.float32),
        scratch_types=[
            pltpu.VMEM((ng, G), jnp.int32),
            pltpu.VMEM((G, EMB), jnp.float32),
            pltpu.SemaphoreType.DMA,
        ],
        compiler_params=pltpu.CompilerParams(use_tc_tiling_on_sc=False),
    )
    def k(tok_hbm, table_hbm, out_hbm, idx_v, rows_v, sem):
        wid = lax.axis_index("s") * NC + lax.axis_index("c")
        pltpu.sync_copy(tok_hbm.at[wid], idx_v)

        def group_body(g, carry):
            pltpu.async_copy(table_hbm.at[idx_v.at[g]], rows_v, sem).wait()

            def scale_body(r, c):
                for j in range(EMB // 16):
                    sl = (r, pl.ds(j * 16, 16))
                    rows_v[sl] = rows_v[sl] * SCALE
                return c

            lax.fori_loop(0, G, scale_body, 0)
            pltpu.sync_copy(rows_v, out_hbm.at[wid, g])
            return carry

        lax.fori_loop(0, ng, group_body, 0)

    return k(tok, table)


def kernel(tokens, table):
    s0, s1 = tokens.shape
    b = s0 * s1
    ng = b // (NW * G)
    tok = tokens.astype(jnp.int32).reshape(NW, ng, G)
    out = _run(tok, table, ng)
    return out.reshape(s0, s1, EMB)


# speedup vs baseline: 1.2044x; 1.2044x over previous
import functools
import jax
import jax.numpy as jnp
from jax import lax
from jax.experimental import pallas as pl
from jax.experimental.pallas import tpu as pltpu
from jax.experimental.pallas import tpu_sc as plsc

EMB = 64
SCALE = 8.0
NC = 2
NS = 16
NW = NC * NS
G = 128
NBUF = 4
D = 2


@functools.partial(jax.jit, static_argnums=(2,))
def _run(tok, table, ng):
    mesh = plsc.VectorSubcoreMesh(core_axis_name="c", subcore_axis_name="s")

    @functools.partial(
        pl.kernel,
        mesh=mesh,
        out_type=jax.ShapeDtypeStruct((NW, ng, G, EMB), jnp.float32),
        scratch_types=[
            pltpu.VMEM((ng, G), jnp.int32),
            pltpu.VMEM((NBUF, G, EMB), jnp.float32),
        ]
        + [pltpu.SemaphoreType.DMA] * (2 * NBUF),
        compiler_params=pltpu.CompilerParams(use_tc_tiling_on_sc=False),
    )
    def k(tok_hbm, table_hbm, out_hbm, idx_v, rows_v, *sems):
        in_sems = sems[:NBUF]
        out_sems = sems[NBUF:]
        wid = lax.axis_index("s") * NC + lax.axis_index("c")
        pltpu.sync_copy(tok_hbm.at[wid], idx_v)

        def gather(g, b):
            return pltpu.async_copy(
                table_hbm.at[idx_v.at[g]], rows_v.at[b], in_sems[b]
            )

        def wait_gather(g, b):
            pltpu.make_async_copy(
                table_hbm.at[idx_v.at[g]], rows_v.at[b], in_sems[b]
            ).wait()

        def put(g, b):
            return pltpu.async_copy(rows_v.at[b], out_hbm.at[wid, g], out_sems[b])

        def wait_put(b):
            pltpu.make_async_copy(
                rows_v.at[b], out_hbm.at[wid, 0], out_sems[b]
            ).wait()

        def scale(b):
            def body(r, c):
                for rr in range(8):
                    row = r * 8 + rr
                    for j in range(EMB // 16):
                        sl = (b, row, pl.ds(j * 16, 16))
                        rows_v[sl] = rows_v[sl] * SCALE
                return c

            lax.fori_loop(0, G // 8, body, 0)

        gather(0, 0)
        gather(1, 1)
        for g in range(D):
            gather(g + D, g + D)
            wait_gather(g, g)
            scale(g)
            put(g, g)

        def steady(t, c):
            for b in range(NBUF):
                g = D + t * NBUF + b
                cb = (D + b) % NBUF
                wait_put(b)
                gather(g + D, b)
                wait_gather(g, cb)
                scale(cb)
                put(g, cb)
            return c

        lax.fori_loop(0, (ng - 2 * D) // NBUF, steady, 0)

        for i in range(D):
            g = ng - D + i
            cb = g % NBUF
            wait_gather(g, cb)
            scale(cb)
            put(g, cb)
        for b in range(NBUF):
            wait_put(b)

    return k(tok, table)


def kernel(tokens, table):
    s0, s1 = tokens.shape
    b = s0 * s1
    ng = b // (NW * G)
    tok = tokens.astype(jnp.int32).reshape(NW, ng, G)
    out = _run(tok, table, ng)
    return out.reshape(s0, s1, EMB)
